# Initial kernel scaffold; baseline (speedup 1.0000x reference)
import functools

import jax
import jax.numpy as jnp
from jax import lax
from jax.experimental import pallas as pl
from jax.experimental.pallas import tpu as pltpu
from jax.experimental.pallas import tpu_sc as plsc

_N, _E, _D, _H, _OUT, _G = 10000, 320000, 128, 128, 64, 128
_NC, _NS = 2, 16
_NW = _NC * _NS
_K = 80
_CPT = _E // _NW // _K
_ROWS_E = _E // _K
_RPS = _N // _NS
_ZR = 125
_RB = 1000


def _make_sc_agg(with_cnt):
  mesh = plsc.VectorSubcoreMesh(core_axis_name="c", subcore_axis_name="s")
  out_type = [jax.ShapeDtypeStruct((_NC, _N, _H), jnp.float32)]
  scratch = [
      pltpu.VMEM((_CPT, _K), jnp.int32),
      pltpu.VMEM((_CPT, _K), jnp.int32),
      pltpu.VMEM((_K, _H), jnp.float32),
      pltpu.VMEM((_K, _H), jnp.float32),
      pltpu.VMEM((_ZR, _H), jnp.float32),
      pltpu.VMEM_SHARED((_N, _H), jnp.float32),
      pltpu.SemaphoreType.DMA,
      pltpu.SemaphoreType.DMA,
  ]
  if with_cnt:
    out_type.append(jax.ShapeDtypeStruct((_NC, _N, 16), jnp.float32))
    scratch += [
        pltpu.VMEM((_K, 16), jnp.float32),
        pltpu.VMEM((_ZR, 16), jnp.float32),
        pltpu.VMEM_SHARED((_N, 16), jnp.float32),
    ]

  @functools.partial(pl.kernel, mesh=mesh, out_type=out_type,
                     scratch_types=scratch)
  def agg(hn_hbm, src_hbm, dst_hbm, *refs):
    if with_cnt:
      (out_hbm, cnt_hbm, src_v, dst_v, rows0, rows1, zbuf, aggm, sem0, sem1,
       ones_v, z16, cntm) = refs
    else:
      (out_hbm, src_v, dst_v, rows0, rows1, zbuf, aggm, sem0, sem1) = refs
    c = lax.axis_index("c")
    s = lax.axis_index("s")

    def zrow(r, _):
      for j in range(_H // 16):
        zbuf[r, pl.ds(j * 16, 16)] = jnp.zeros((16,), jnp.float32)
      return 0
    lax.fori_loop(0, _ZR, zrow, 0)
    for t in range(_RPS // _ZR):
      pltpu.sync_copy(zbuf, aggm.at[pl.ds(s * _RPS + t * _ZR, _ZR)])
    if with_cnt:
      def orow(r, _):
        ones_v[r, :] = jnp.ones((16,), jnp.float32)
        z16[r, :] = jnp.zeros((16,), jnp.float32)
        return 0
      lax.fori_loop(0, _ZR, orow, 0)
      for t in range(_RPS // _ZR):
        pltpu.sync_copy(z16, cntm.at[pl.ds(s * _RPS + t * _ZR, _ZR)])
    plsc.subcore_barrier()

    base = (c * _NS + s) * _CPT
    pltpu.sync_copy(src_hbm.at[pl.ds(base, _CPT)], src_v)
    pltpu.sync_copy(dst_hbm.at[pl.ds(base, _CPT)], dst_v)

    pltpu.make_async_copy(hn_hbm.at[src_v.at[0]], rows0, sem0).start()
    pltpu.make_async_copy(hn_hbm.at[src_v.at[1]], rows1, sem1).start()

    def body(i, _):
      g = i * 2
      for b, (rb, sb) in enumerate(((rows0, sem0), (rows1, sem1))):
        ch = g + b
        pltpu.make_async_copy(hn_hbm.at[src_v.at[ch]], rb, sb).wait()
        pltpu.sync_copy(rb, aggm.at[dst_v.at[ch]], add=True)
        if with_cnt:
          pltpu.sync_copy(ones_v, cntm.at[dst_v.at[ch]], add=True)
        nxt = ch + 2

        @pl.when(nxt < _CPT)
        def _start():
          pltpu.make_async_copy(hn_hbm.at[src_v.at[nxt]], rb, sb).start()
      return 0
    lax.fori_loop(0, _CPT // 2, body, 0)

    last = _CPT - 1
    pltpu.make_async_copy(hn_hbm.at[src_v.at[last]], rows0, sem0).wait()
    pltpu.sync_copy(rows0, aggm.at[dst_v.at[last]], add=True)
    if with_cnt:
      pltpu.sync_copy(ones_v, cntm.at[dst_v.at[last]], add=True)
    plsc.subcore_barrier()

    pltpu.sync_copy(aggm.at[pl.ds(s * _RPS, _RPS)],
                    out_hbm.at[c, pl.ds(s * _RPS, _RPS)])
    if with_cnt:
      pltpu.sync_copy(cntm.at[pl.ds(s * _RPS, _RPS)],
                      cnt_hbm.at[c, pl.ds(s * _RPS, _RPS)])

  return agg


_sc_agg_cnt = _make_sc_agg(True)
_sc_agg = _make_sc_agg(False)


def _dot(a, b):
  return jnp.dot(a, b, preferred_element_type=jnp.float32,
                 precision=lax.Precision.HIGHEST)


def _lin0_body(x_ref, wn_ref, wr_ref, b_ref, hn_ref, hr_ref):
  xb = x_ref[...]
  hn_ref[...] = _dot(xb, wn_ref[...])
  hr_ref[...] = _dot(xb, wr_ref[...]) + b_ref[...]


def _comb_body(p_ref, c_ref, hr_ref, wn_ref, wr_ref, b_ref, hn_ref, hro_ref):
  agg = p_ref[0] + p_ref[1]
  cnt = c_ref[0] + c_ref[1]
  inv = 1.0 / jnp.maximum(cnt[:, 0:1], 1.0)
  h = jnp.maximum(agg * inv + hr_ref[...], 0.0)
  hn_ref[...] = _dot(h, wn_ref[...])
  hro_ref[...] = _dot(h, wr_ref[...]) + b_ref[...]


def _pool_body(p_ref, c_ref, hr_ref, batch_ref, sums_ref, cnts_ref):
  i = pl.program_id(0)
  agg = p_ref[0] + p_ref[1]
  cnt = c_ref[0] + c_ref[1]
  inv = 1.0 / jnp.maximum(cnt[:, 0:1], 1.0)
  h = jnp.maximum(agg * inv + hr_ref[...], 0.0)
  seg = batch_ref[...]
  m = (seg == lax.broadcasted_iota(jnp.int32, (_RB, _G), 1)
       ).astype(jnp.float32)
  dn = (((0,), (0,)), ((), ()))
  sums_blk = lax.dot_general(m, h, dimension_numbers=dn,
                             preferred_element_type=jnp.float32,
                             precision=lax.Precision.HIGHEST)
  cnts_blk = lax.dot_general(m, jnp.ones_like(h), dimension_numbers=dn,
                             preferred_element_type=jnp.float32,
                             precision=lax.Precision.HIGHEST)

  @pl.when(i == 0)
  def _init():
    sums_ref[...] = jnp.zeros_like(sums_ref)
    cnts_ref[...] = jnp.zeros_like(cnts_ref)
  sums_ref[...] += sums_blk
  cnts_ref[...] += cnts_blk


def _mlp_body(sums_ref, cnts_ref, w1_ref, b1_ref, w2_ref, b2_ref, out_ref):
  pooled = sums_ref[...] / jnp.maximum(cnts_ref[...], 1.0)
  t = jnp.maximum(_dot(pooled, w1_ref[...]) + b1_ref[...], 0.0)
  out_ref[...] = _dot(t, w2_ref[...]) + b2_ref[...]


def _row_spec(shape):
  if len(shape) == 2:
    return pl.BlockSpec((_RB, shape[1]), lambda i: (i, 0))
  return pl.BlockSpec((shape[0], _RB, shape[2]), lambda i: (0, i, 0))


def _full_spec(shape):
  zero = tuple(0 for _ in shape)
  return pl.BlockSpec(shape, lambda i=0, _z=zero: _z)


def _lin0(x, wn, wr, b):
  return pl.pallas_call(
      _lin0_body,
      grid=(_N // _RB,),
      in_specs=[_row_spec((_N, _D)), _full_spec((_D, _H)),
                _full_spec((_D, _H)), _full_spec((1, _H))],
      out_specs=[_row_spec((_N, _H)), _row_spec((_N, _H))],
      out_shape=[jax.ShapeDtypeStruct((_N, _H), jnp.float32)] * 2,
  )(x, wn, wr, b)


def _comb(p, c, hr, wn, wr, b):
  return pl.pallas_call(
      _comb_body,
      grid=(_N // _RB,),
      in_specs=[_row_spec((_NC, _N, _H)), _row_spec((_NC, _N, 16)),
                _row_spec((_N, _H)), _full_spec((_H, _H)),
                _full_spec((_H, _H)), _full_spec((1, _H))],
      out_specs=[_row_spec((_N, _H)), _row_spec((_N, _H))],
      out_shape=[jax.ShapeDtypeStruct((_N, _H), jnp.float32)] * 2,
  )(p, c, hr, wn, wr, b)


def _pool(p, c, hr, batch2):
  return pl.pallas_call(
      _pool_body,
      grid=(_N // _RB,),
      in_specs=[_row_spec((_NC, _N, _H)), _row_spec((_NC, _N, 16)),
                _row_spec((_N, _H)), _row_spec((_N, 1))],
      out_specs=[_full_spec((_G, _H)), _full_spec((_G, _H))],
      out_shape=[jax.ShapeDtypeStruct((_G, _H), jnp.float32)] * 2,
  )(p, c, hr, batch2)


def _mlp(sums, cnts, w1, b1, w2, b2):
  return pl.pallas_call(
      _mlp_body,
      in_specs=[_full_spec((_G, _H)), _full_spec((_G, _H)),
                _full_spec((_H, _H)), _full_spec((1, _H)),
                _full_spec((_H, _OUT)), _full_spec((1, _OUT))],
      out_specs=_full_spec((_G, _OUT)),
      out_shape=jax.ShapeDtypeStruct((_G, _OUT), jnp.float32),
  )(sums, cnts, w1, b1, w2, b2)


def kernel(x, edge_index, batch, W_neigh_0, W_root_0, b_0, W_neigh_1,
           W_root_1, b_1, W_neigh_2, W_root_2, b_2, W_neigh_3, W_root_3,
           b_3, fc1_W, fc1_b, fc2_W, fc2_b):
  src = edge_index[0].reshape(_ROWS_E, _K)
  dst = edge_index[1].reshape(_ROWS_E, _K)
  batch2 = batch.reshape(_N, 1)

  hn, hr = _lin0(x, W_neigh_0, W_root_0, b_0.reshape(1, _H))
  p, cnt = _sc_agg_cnt(hn, src, dst)
  hn, hr = _comb(p, cnt, hr, W_neigh_1, W_root_1, b_1.reshape(1, _H))
  p = _sc_agg(hn, src, dst)
  hn, hr = _comb(p, cnt, hr, W_neigh_2, W_root_2, b_2.reshape(1, _H))
  p = _sc_agg(hn, src, dst)
  hn, hr = _comb(p, cnt, hr, W_neigh_3, W_root_3, b_3.reshape(1, _H))
  p = _sc_agg(hn, src, dst)
  sums, cnts = _pool(p, cnt, hr, batch2)
  return _mlp(sums, cnts, fc1_W, fc1_b.reshape(1, _H),
              fc2_W, fc2_b.reshape(1, _OUT))


# speedup vs baseline: 7.3063x; 7.3063x over previous
import functools

import jax
import jax.numpy as jnp
from jax import lax
from jax.experimental import pallas as pl
from jax.experimental.pallas import tpu as pltpu
from jax.experimental.pallas import tpu_sc as plsc

_N, _E, _D, _H, _OUT, _G = 10000, 320000, 128, 128, 64, 128
_HH = _H // 2
_NC, _NS = 2, 16
_NW = _NC * _NS
_K = 80
_CPT = _E // _NW // _K
_NP = 10240
_RPS = _NP // _NS
_ZR = 128
_RB = 1000


def _make_sc_agg(with_cnt):
  mesh = plsc.VectorSubcoreMesh(core_axis_name="c", subcore_axis_name="s")
  out_type = [jax.ShapeDtypeStruct((_NC, _NP, _HH), jnp.float32)] * 2
  scratch = [
      pltpu.VMEM((_CPT, _K), jnp.int32),
      pltpu.VMEM((_CPT, _K), jnp.int32),
      pltpu.VMEM((_K, _HH), jnp.float32),
      pltpu.VMEM((_K, _HH), jnp.float32),
      pltpu.VMEM((_ZR, _HH), jnp.float32),
      pltpu.VMEM_SHARED((_NP, _HH), jnp.float32),
      pltpu.SemaphoreType.DMA,
      pltpu.SemaphoreType.DMA,
  ]
  if with_cnt:
    out_type.append(jax.ShapeDtypeStruct((_NC, _NP, 16), jnp.float32))
    scratch += [
        pltpu.VMEM((_K, 16), jnp.float32),
        pltpu.VMEM((_ZR, 16), jnp.float32),
        pltpu.VMEM_SHARED((_NP, 16), jnp.float32),
    ]

  @functools.partial(
      pl.kernel, mesh=mesh, out_type=out_type, scratch_types=scratch,
      compiler_params=pltpu.CompilerParams(use_tc_tiling_on_sc=False))
  def agg(hn_lo_hbm, hn_hi_hbm, src_hbm, dst_hbm, *refs):
    if with_cnt:
      (out_lo_hbm, out_hi_hbm, cnt_hbm, src_v, dst_v, rows0, rows1, zbuf,
       aggm, sem0, sem1, ones_v, z16, cntm) = refs
    else:
      (out_lo_hbm, out_hi_hbm, src_v, dst_v, rows0, rows1, zbuf,
       aggm, sem0, sem1) = refs
    c = lax.axis_index("c")
    s = lax.axis_index("s")
    w = c * _NS + s

    def zrow(r, _):
      for j in range(_HH // 16):
        zbuf[r, pl.ds(j * 16, 16)] = jnp.zeros((16,), jnp.float32)
      return 0
    lax.fori_loop(0, _ZR, zrow, 0)
    if with_cnt:
      def orow(r, _):
        ones_v[r, :] = jnp.ones((16,), jnp.float32)
        return 0
      lax.fori_loop(0, _K, orow, 0)
      def z16row(r, _):
        z16[r, :] = jnp.zeros((16,), jnp.float32)
        return 0
      lax.fori_loop(0, _ZR, z16row, 0)

    pltpu.sync_copy(src_hbm.at[w], src_v)
    pltpu.sync_copy(dst_hbm.at[w], dst_v)

    for half, (hn_hbm, out_hbm) in enumerate(
        ((hn_lo_hbm, out_lo_hbm), (hn_hi_hbm, out_hi_hbm))):
      do_cnt = with_cnt and half == 0
      for t in range(_RPS // _ZR):
        pltpu.sync_copy(zbuf, aggm.at[pl.ds(s * _RPS + t * _ZR, _ZR)])
      if do_cnt:
        for t in range(_RPS // _ZR):
          pltpu.sync_copy(z16, cntm.at[pl.ds(s * _RPS + t * _ZR, _ZR)])
      plsc.subcore_barrier()

      pltpu.make_async_copy(hn_hbm.at[src_v.at[0]], rows0, sem0).start()
      pltpu.make_async_copy(hn_hbm.at[src_v.at[1]], rows1, sem1).start()

      def body(i, _):
        g = i * 2
        for b, (rb, sb) in enumerate(((rows0, sem0), (rows1, sem1))):
          ch = g + b
          pltpu.make_async_copy(hn_hbm.at[src_v.at[ch]], rb, sb).wait()
          pltpu.sync_copy(rb, aggm.at[dst_v.at[ch]], add=True)
          if do_cnt:
            pltpu.sync_copy(ones_v, cntm.at[dst_v.at[ch]], add=True)
          nxt = ch + 2

          @pl.when(nxt < _CPT)
          def _start():
            pltpu.make_async_copy(hn_hbm.at[src_v.at[nxt]], rb, sb).start()
        return 0
      lax.fori_loop(0, _CPT // 2, body, 0)

      last = _CPT - 1
      pltpu.make_async_copy(hn_hbm.at[src_v.at[last]], rows0, sem0).wait()
      pltpu.sync_copy(rows0, aggm.at[dst_v.at[last]], add=True)
      if do_cnt:
        pltpu.sync_copy(ones_v, cntm.at[dst_v.at[last]], add=True)
      plsc.subcore_barrier()

      pltpu.sync_copy(aggm.at[pl.ds(s * _RPS, _RPS)],
                      out_hbm.at[c, pl.ds(s * _RPS, _RPS)])
      if do_cnt:
        pltpu.sync_copy(cntm.at[pl.ds(s * _RPS, _RPS)],
                        cnt_hbm.at[c, pl.ds(s * _RPS, _RPS)])

  return agg


_sc_agg_cnt = _make_sc_agg(True)
_sc_agg = _make_sc_agg(False)


def _dot(a, b):
  return jnp.dot(a, b, preferred_element_type=jnp.float32,
                 precision=lax.Precision.HIGHEST)


def _lin0_body(x_ref, wn_ref, wr_ref, b_ref, hnl_ref, hnh_ref, hr_ref):
  xb = x_ref[...]
  hn = _dot(xb, wn_ref[...])
  hnl_ref[...] = hn[:, :_HH]
  hnh_ref[...] = hn[:, _HH:]
  hr_ref[...] = _dot(xb, wr_ref[...]) + b_ref[...]


def _combine(pl_ref, ph_ref, c_ref, hr_ref):
  agg = jnp.concatenate([pl_ref[0] + pl_ref[1], ph_ref[0] + ph_ref[1]],
                        axis=1)
  cnt = c_ref[0] + c_ref[1]
  inv = 1.0 / jnp.maximum(cnt[:, 0:1], 1.0)
  return jnp.maximum(agg * inv + hr_ref[...], 0.0)


def _comb_body(pl_ref, ph_ref, c_ref, hr_ref, wn_ref, wr_ref, b_ref,
               hnl_ref, hnh_ref, hro_ref):
  h = _combine(pl_ref, ph_ref, c_ref, hr_ref)
  hn = _dot(h, wn_ref[...])
  hnl_ref[...] = hn[:, :_HH]
  hnh_ref[...] = hn[:, _HH:]
  hro_ref[...] = _dot(h, wr_ref[...]) + b_ref[...]


def _pool_body(pl_ref, ph_ref, c_ref, hr_ref, batch_ref, sums_ref, cnts_ref):
  i = pl.program_id(0)
  h = _combine(pl_ref, ph_ref, c_ref, hr_ref)
  seg = batch_ref[...]
  m = (seg == lax.broadcasted_iota(jnp.int32, (_RB, _G), 1)
       ).astype(jnp.float32)
  dn = (((0,), (0,)), ((), ()))
  sums_blk = lax.dot_general(m, h, dimension_numbers=dn,
                             preferred_element_type=jnp.float32,
                             precision=lax.Precision.HIGHEST)
  cnts_blk = lax.dot_general(m, jnp.ones_like(h), dimension_numbers=dn,
                             preferred_element_type=jnp.float32,
                             precision=lax.Precision.HIGHEST)

  @pl.when(i == 0)
  def _init():
    sums_ref[...] = jnp.zeros_like(sums_ref)
    cnts_ref[...] = jnp.zeros_like(cnts_ref)
  sums_ref[...] += sums_blk
  cnts_ref[...] += cnts_blk


def _mlp_body(sums_ref, cnts_ref, w1_ref, b1_ref, w2_ref, b2_ref, out_ref):
  pooled = sums_ref[...] / jnp.maximum(cnts_ref[...], 1.0)
  t = jnp.maximum(_dot(pooled, w1_ref[...]) + b1_ref[...], 0.0)
  out_ref[...] = _dot(t, w2_ref[...]) + b2_ref[...]


def _row_spec(shape):
  if len(shape) == 2:
    return pl.BlockSpec((_RB, shape[1]), lambda i: (i, 0))
  return pl.BlockSpec((shape[0], _RB, shape[2]), lambda i: (0, i, 0))


def _full_spec(shape):
  zero = tuple(0 for _ in shape)
  return pl.BlockSpec(shape, lambda i=0, _z=zero: _z)


_HN_SHAPES = [jax.ShapeDtypeStruct((_N, _HH), jnp.float32)] * 2


def _lin0(x, wn, wr, b):
  return pl.pallas_call(
      _lin0_body,
      grid=(_N // _RB,),
      in_specs=[_row_spec((_N, _D)), _full_spec((_D, _H)),
                _full_spec((_D, _H)), _full_spec((1, _H))],
      out_specs=[_row_spec((_N, _HH)), _row_spec((_N, _HH)),
                 _row_spec((_N, _H))],
      out_shape=_HN_SHAPES + [jax.ShapeDtypeStruct((_N, _H), jnp.float32)],
  )(x, wn, wr, b)


def _comb(plo, phi, c, hr, wn, wr, b):
  return pl.pallas_call(
      _comb_body,
      grid=(_N // _RB,),
      in_specs=[_row_spec((_NC, _NP, _HH)), _row_spec((_NC, _NP, _HH)),
                _row_spec((_NC, _NP, 16)), _row_spec((_N, _H)),
                _full_spec((_H, _H)), _full_spec((_H, _H)),
                _full_spec((1, _H))],
      out_specs=[_row_spec((_N, _HH)), _row_spec((_N, _HH)),
                 _row_spec((_N, _H))],
      out_shape=_HN_SHAPES + [jax.ShapeDtypeStruct((_N, _H), jnp.float32)],
  )(plo, phi, c, hr, wn, wr, b)


def _pool(plo, phi, c, hr, batch2):
  return pl.pallas_call(
      _pool_body,
      grid=(_N // _RB,),
      in_specs=[_row_spec((_NC, _NP, _HH)), _row_spec((_NC, _NP, _HH)),
                _row_spec((_NC, _NP, 16)), _row_spec((_N, _H)),
                _row_spec((_N, 1))],
      out_specs=[_full_spec((_G, _H)), _full_spec((_G, _H))],
      out_shape=[jax.ShapeDtypeStruct((_G, _H), jnp.float32)] * 2,
  )(plo, phi, c, hr, batch2)


def _mlp(sums, cnts, w1, b1, w2, b2):
  return pl.pallas_call(
      _mlp_body,
      in_specs=[_full_spec((_G, _H)), _full_spec((_G, _H)),
                _full_spec((_H, _H)), _full_spec((1, _H)),
                _full_spec((_H, _OUT)), _full_spec((1, _OUT))],
      out_specs=_full_spec((_G, _OUT)),
      out_shape=jax.ShapeDtypeStruct((_G, _OUT), jnp.float32),
  )(sums, cnts, w1, b1, w2, b2)


def _dbg_agg(hnl, hnh, src, dst):
  sf = src.reshape(-1)
  df = dst.reshape(-1)
  hn = jnp.concatenate([hnl, hnh], axis=1)
  agg = jnp.zeros((_NP, _H), jnp.float32).at[df].add(hn[sf])
  z = jnp.zeros((1, _NP, _HH), jnp.float32)
  plo = jnp.concatenate([agg[None, :, :_HH], z], axis=0)
  phi = jnp.concatenate([agg[None, :, _HH:], z], axis=0)
  cnt = jnp.zeros((_NP,), jnp.float32).at[df].add(1.0)
  cntm = jnp.concatenate([jnp.broadcast_to(cnt[None, :, None], (1, _NP, 16)),
                          jnp.zeros((1, _NP, 16), jnp.float32)], axis=0)
  return plo, phi, cntm


def kernel(x, edge_index, batch, W_neigh_0, W_root_0, b_0, W_neigh_1,
           W_root_1, b_1, W_neigh_2, W_root_2, b_2, W_neigh_3, W_root_3,
           b_3, fc1_W, fc1_b, fc2_W, fc2_b):
  src = edge_index[0].reshape(_NW, _CPT, _K)
  dst = edge_index[1].reshape(_NW, _CPT, _K)
  batch2 = batch.reshape(_N, 1)

  hnl, hnh, hr = _lin0(x, W_neigh_0, W_root_0, b_0.reshape(1, _H))
  plo, phi, cnt = _sc_agg_cnt(hnl, hnh, src, dst)
  hnl, hnh, hr = _comb(plo, phi, cnt, hr, W_neigh_1, W_root_1,
                       b_1.reshape(1, _H))
  plo, phi = _sc_agg(hnl, hnh, src, dst)
  hnl, hnh, hr = _comb(plo, phi, cnt, hr, W_neigh_2, W_root_2,
                       b_2.reshape(1, _H))
  plo, phi = _sc_agg(hnl, hnh, src, dst)
  hnl, hnh, hr = _comb(plo, phi, cnt, hr, W_neigh_3, W_root_3,
                       b_3.reshape(1, _H))
  plo, phi = _sc_agg(hnl, hnh, src, dst)
  sums, cnts = _pool(plo, phi, cnt, hr, batch2)
  return _mlp(sums, cnts, fc1_W, fc1_b.reshape(1, _H),
              fc2_W, fc2_b.reshape(1, _OUT))


# speedup vs baseline: 8.1896x; 1.1209x over previous
import functools

import jax
import jax.numpy as jnp
from jax import lax
from jax.experimental import pallas as pl
from jax.experimental.pallas import tpu as pltpu
from jax.experimental.pallas import tpu_sc as plsc

_N, _E, _D, _H, _OUT, _G = 10000, 320000, 128, 128, 64, 128
_HH = _H // 2
_NC, _NS = 2, 16
_NW = _NC * _NS
_K = 80
_CPT = _E // _NW // _K
_NP = 10240
_RPS = _NP // _NS
_ZR = 128
_RB = 1000


def _make_sc_agg(with_cnt):
  mesh = plsc.VectorSubcoreMesh(core_axis_name="c", subcore_axis_name="s")
  out_type = [jax.ShapeDtypeStruct((_NC, _NP, _HH), jnp.float32)] * 2
  scratch = [
      pltpu.VMEM((_CPT, _K), jnp.int32),
      pltpu.VMEM((_CPT, _K), jnp.int32),
      pltpu.VMEM((4, _K, _HH), jnp.float32),
      pltpu.VMEM((_ZR, _HH), jnp.float32),
      pltpu.VMEM_SHARED((_NP, _HH), jnp.float32),
  ] + [pltpu.SemaphoreType.DMA] * 8
  if with_cnt:
    out_type.append(jax.ShapeDtypeStruct((_NC, _NP, 16), jnp.float32))
    scratch += [
        pltpu.VMEM((_K, 16), jnp.float32),
        pltpu.VMEM((_ZR, 16), jnp.float32),
        pltpu.VMEM_SHARED((_NP, 16), jnp.float32),
    ]

  @functools.partial(
      pl.kernel, mesh=mesh, out_type=out_type, scratch_types=scratch,
      compiler_params=pltpu.CompilerParams(use_tc_tiling_on_sc=False))
  def agg(hn_lo_hbm, hn_hi_hbm, src_hbm, dst_hbm, *refs):
    if with_cnt:
      (out_lo_hbm, out_hi_hbm, cnt_hbm, src_v, dst_v, rows, zbuf,
       aggm, g0, g1, g2, g3, s0, s1, s2, s3, ones_v, z16, cntm) = refs
    else:
      (out_lo_hbm, out_hi_hbm, src_v, dst_v, rows, zbuf,
       aggm, g0, g1, g2, g3, s0, s1, s2, s3) = refs
    gsem = (g0, g1, g2, g3)
    ssem = (s0, s1, s2, s3)
    c = lax.axis_index("c")
    s = lax.axis_index("s")
    w = c * _NS + s

    def zrow(r, _):
      for j in range(_HH // 16):
        zbuf[r, pl.ds(j * 16, 16)] = jnp.zeros((16,), jnp.float32)
      return 0
    lax.fori_loop(0, _ZR, zrow, 0)
    if with_cnt:
      def orow(r, _):
        ones_v[r, :] = jnp.ones((16,), jnp.float32)
        return 0
      lax.fori_loop(0, _K, orow, 0)
      def z16row(r, _):
        z16[r, :] = jnp.zeros((16,), jnp.float32)
        return 0
      lax.fori_loop(0, _ZR, z16row, 0)

    pltpu.sync_copy(src_hbm.at[w], src_v)
    pltpu.sync_copy(dst_hbm.at[w], dst_v)

    for half, (hn_hbm, out_hbm) in enumerate(
        ((hn_lo_hbm, out_lo_hbm), (hn_hi_hbm, out_hi_hbm))):
      do_cnt = with_cnt and half == 0
      for t in range(_RPS // _ZR):
        pltpu.sync_copy(zbuf, aggm.at[pl.ds(s * _RPS + t * _ZR, _ZR)])
      if do_cnt:
        for t in range(_RPS // _ZR):
          pltpu.sync_copy(z16, cntm.at[pl.ds(s * _RPS + t * _ZR, _ZR)])
      plsc.subcore_barrier()

      def gath(ch, b):
        return pltpu.make_async_copy(hn_hbm.at[src_v.at[ch]], rows.at[b],
                                     gsem[b])

      def scat(ch, b):
        return pltpu.make_async_copy(rows.at[b], aggm.at[dst_v.at[ch]],
                                     ssem[b])

      gath(0, 0).start()
      gath(1, 1).start()

      def body(i, _):
        g4 = i * 4
        for b in range(4):
          ch = g4 + b
          gath(ch, b).wait()
          scat(ch, b).start(add=True)
          if do_cnt:
            pltpu.sync_copy(ones_v, cntm.at[dst_v.at[ch]], add=True)
          nxt = ch + 2
          nb = (b + 2) % 4

          @pl.when(nxt < _CPT)
          def _start():
            @pl.when(nxt >= 4)
            def _drain():
              scat(nxt - 4, nb).wait()
            gath(nxt, nb).start()
        return 0
      lax.fori_loop(0, _CPT // 4, body, 0)

      last = _CPT - 1
      gath(last, 0).wait()
      scat(last, 0).start(add=True)
      if do_cnt:
        pltpu.sync_copy(ones_v, cntm.at[dst_v.at[last]], add=True)
      scat(last - 3, 1).wait()
      scat(last - 2, 2).wait()
      scat(last - 1, 3).wait()
      scat(last, 0).wait()
      plsc.subcore_barrier()

      pltpu.sync_copy(aggm.at[pl.ds(s * _RPS, _RPS)],
                      out_hbm.at[c, pl.ds(s * _RPS, _RPS)])
      if do_cnt:
        pltpu.sync_copy(cntm.at[pl.ds(s * _RPS, _RPS)],
                        cnt_hbm.at[c, pl.ds(s * _RPS, _RPS)])

  return agg


_sc_agg_cnt = _make_sc_agg(True)
_sc_agg = _make_sc_agg(False)


def _dot(a, b):
  return jnp.dot(a, b, preferred_element_type=jnp.float32)


def _lin0_body(x_ref, wn_ref, wr_ref, b_ref, hnl_ref, hnh_ref, hr_ref):
  xb = x_ref[...]
  hn = _dot(xb, wn_ref[...])
  hnl_ref[...] = hn[:, :_HH]
  hnh_ref[...] = hn[:, _HH:]
  hr_ref[...] = _dot(xb, wr_ref[...]) + b_ref[...]


def _combine(pl_ref, ph_ref, c_ref, hr_ref):
  agg = jnp.concatenate([pl_ref[0] + pl_ref[1], ph_ref[0] + ph_ref[1]],
                        axis=1)
  cnt = c_ref[0] + c_ref[1]
  inv = 1.0 / jnp.maximum(cnt[:, 0:1], 1.0)
  return jnp.maximum(agg * inv + hr_ref[...], 0.0)


def _comb_body(pl_ref, ph_ref, c_ref, hr_ref, wn_ref, wr_ref, b_ref,
               hnl_ref, hnh_ref, hro_ref):
  h = _combine(pl_ref, ph_ref, c_ref, hr_ref)
  hn = _dot(h, wn_ref[...])
  hnl_ref[...] = hn[:, :_HH]
  hnh_ref[...] = hn[:, _HH:]
  hro_ref[...] = _dot(h, wr_ref[...]) + b_ref[...]


def _pool_body(pl_ref, ph_ref, c_ref, hr_ref, batch_ref, sums_ref, cnts_ref):
  i = pl.program_id(0)
  h = _combine(pl_ref, ph_ref, c_ref, hr_ref)
  seg = batch_ref[...]
  m = (seg == lax.broadcasted_iota(jnp.int32, (_RB, _G), 1)
       ).astype(jnp.float32)
  dn = (((0,), (0,)), ((), ()))
  sums_blk = lax.dot_general(m, h, dimension_numbers=dn,
                             preferred_element_type=jnp.float32,
                             precision=lax.Precision.HIGHEST)
  cnts_blk = lax.dot_general(m, jnp.ones_like(h), dimension_numbers=dn,
                             preferred_element_type=jnp.float32,
                             precision=lax.Precision.HIGHEST)

  @pl.when(i == 0)
  def _init():
    sums_ref[...] = jnp.zeros_like(sums_ref)
    cnts_ref[...] = jnp.zeros_like(cnts_ref)
  sums_ref[...] += sums_blk
  cnts_ref[...] += cnts_blk


def _mlp_body(sums_ref, cnts_ref, w1_ref, b1_ref, w2_ref, b2_ref, out_ref):
  pooled = sums_ref[...] / jnp.maximum(cnts_ref[...], 1.0)
  t = jnp.maximum(_dot(pooled, w1_ref[...]) + b1_ref[...], 0.0)
  out_ref[...] = _dot(t, w2_ref[...]) + b2_ref[...]


def _row_spec(shape):
  if len(shape) == 2:
    return pl.BlockSpec((_RB, shape[1]), lambda i: (i, 0))
  return pl.BlockSpec((shape[0], _RB, shape[2]), lambda i: (0, i, 0))


def _full_spec(shape):
  zero = tuple(0 for _ in shape)
  return pl.BlockSpec(shape, lambda i=0, _z=zero: _z)


_HN_SHAPES = [jax.ShapeDtypeStruct((_N, _HH), jnp.float32)] * 2


def _lin0(x, wn, wr, b):
  return pl.pallas_call(
      _lin0_body,
      grid=(_N // _RB,),
      in_specs=[_row_spec((_N, _D)), _full_spec((_D, _H)),
                _full_spec((_D, _H)), _full_spec((1, _H))],
      out_specs=[_row_spec((_N, _HH)), _row_spec((_N, _HH)),
                 _row_spec((_N, _H))],
      out_shape=_HN_SHAPES + [jax.ShapeDtypeStruct((_N, _H), jnp.float32)],
  )(x, wn, wr, b)


def _comb(plo, phi, c, hr, wn, wr, b):
  return pl.pallas_call(
      _comb_body,
      grid=(_N // _RB,),
      in_specs=[_row_spec((_NC, _NP, _HH)), _row_spec((_NC, _NP, _HH)),
                _row_spec((_NC, _NP, 16)), _row_spec((_N, _H)),
                _full_spec((_H, _H)), _full_spec((_H, _H)),
                _full_spec((1, _H))],
      out_specs=[_row_spec((_N, _HH)), _row_spec((_N, _HH)),
                 _row_spec((_N, _H))],
      out_shape=_HN_SHAPES + [jax.ShapeDtypeStruct((_N, _H), jnp.float32)],
  )(plo, phi, c, hr, wn, wr, b)


def _pool(plo, phi, c, hr, batch2):
  return pl.pallas_call(
      _pool_body,
      grid=(_N // _RB,),
      in_specs=[_row_spec((_NC, _NP, _HH)), _row_spec((_NC, _NP, _HH)),
                _row_spec((_NC, _NP, 16)), _row_spec((_N, _H)),
                _row_spec((_N, 1))],
      out_specs=[_full_spec((_G, _H)), _full_spec((_G, _H))],
      out_shape=[jax.ShapeDtypeStruct((_G, _H), jnp.float32)] * 2,
  )(plo, phi, c, hr, batch2)


def _mlp(sums, cnts, w1, b1, w2, b2):
  return pl.pallas_call(
      _mlp_body,
      in_specs=[_full_spec((_G, _H)), _full_spec((_G, _H)),
                _full_spec((_H, _H)), _full_spec((1, _H)),
                _full_spec((_H, _OUT)), _full_spec((1, _OUT))],
      out_specs=_full_spec((_G, _OUT)),
      out_shape=jax.ShapeDtypeStruct((_G, _OUT), jnp.float32),
  )(sums, cnts, w1, b1, w2, b2)


def _dbg_agg(hnl, hnh, src, dst):
  sf = src.reshape(-1)
  df = dst.reshape(-1)
  hn = jnp.concatenate([hnl, hnh], axis=1)
  agg = jnp.zeros((_NP, _H), jnp.float32).at[df].add(hn[sf])
  z = jnp.zeros((1, _NP, _HH), jnp.float32)
  plo = jnp.concatenate([agg[None, :, :_HH], z], axis=0)
  phi = jnp.concatenate([agg[None, :, _HH:], z], axis=0)
  cnt = jnp.zeros((_NP,), jnp.float32).at[df].add(1.0)
  cntm = jnp.concatenate([jnp.broadcast_to(cnt[None, :, None], (1, _NP, 16)),
                          jnp.zeros((1, _NP, 16), jnp.float32)], axis=0)
  return plo, phi, cntm


def kernel(x, edge_index, batch, W_neigh_0, W_root_0, b_0, W_neigh_1,
           W_root_1, b_1, W_neigh_2, W_root_2, b_2, W_neigh_3, W_root_3,
           b_3, fc1_W, fc1_b, fc2_W, fc2_b):
  src = edge_index[0].reshape(_NW, _CPT, _K)
  dst = edge_index[1].reshape(_NW, _CPT, _K)
  batch2 = batch.reshape(_N, 1)

  hnl, hnh, hr = _lin0(x, W_neigh_0, W_root_0, b_0.reshape(1, _H))
  plo, phi, cnt = _sc_agg_cnt(hnl, hnh, src, dst)
  hnl, hnh, hr = _comb(plo, phi, cnt, hr, W_neigh_1, W_root_1,
                       b_1.reshape(1, _H))
  plo, phi = _sc_agg(hnl, hnh, src, dst)
  hnl, hnh, hr = _comb(plo, phi, cnt, hr, W_neigh_2, W_root_2,
                       b_2.reshape(1, _H))
  plo, phi = _sc_agg(hnl, hnh, src, dst)
  hnl, hnh, hr = _comb(plo, phi, cnt, hr, W_neigh_3, W_root_3,
                       b_3.reshape(1, _H))
  plo, phi = _sc_agg(hnl, hnh, src, dst)
  sums, cnts = _pool(plo, phi, cnt, hr, batch2)
  return _mlp(sums, cnts, fc1_W, fc1_b.reshape(1, _H),
              fc2_W, fc2_b.reshape(1, _OUT))


# speedup vs baseline: 9.0671x; 1.1072x over previous
import functools

import jax
import jax.numpy as jnp
from jax import lax
from jax.experimental import pallas as pl
from jax.experimental.pallas import tpu as pltpu
from jax.experimental.pallas import tpu_sc as plsc

_N, _E, _D, _H, _OUT, _G = 10000, 320000, 128, 128, 64, 128
_HH = _H // 2
_NC, _NS = 2, 16
_NW = _NC * _NS
_K = 128
_EP = 327680
_CPT = _EP // _NW // _K
_NP = 10240
_RPS = _NP // _NS
_ZR = 128
_RB = 1000
_NB = 4


def _make_sc_agg(with_cnt):
  mesh = plsc.VectorSubcoreMesh(core_axis_name="c", subcore_axis_name="s")
  out_type = [jax.ShapeDtypeStruct((_NC, _NP, _HH), jnp.float32)] * 2
  scratch = [
      pltpu.VMEM((_CPT, _K), jnp.int32),
      pltpu.VMEM((_CPT, _K), jnp.int32),
      pltpu.VMEM((_NB, _K, _HH), jnp.float32),
      pltpu.VMEM((_ZR, _HH), jnp.float32),
      pltpu.VMEM_SHARED((_NP, _HH), jnp.float32),
  ] + [pltpu.SemaphoreType.DMA] * (2 * _NB)
  if with_cnt:
    out_type.append(jax.ShapeDtypeStruct((_NC, _NP, 16), jnp.float32))
    scratch += [
        pltpu.VMEM((_K, 16), jnp.float32),
        pltpu.VMEM((_ZR, 16), jnp.float32),
        pltpu.VMEM_SHARED((_NP, 16), jnp.float32),
    ]

  @functools.partial(
      pl.kernel, mesh=mesh, out_type=out_type, scratch_types=scratch,
      compiler_params=pltpu.CompilerParams(use_tc_tiling_on_sc=False))
  def agg(hn_lo_hbm, hn_hi_hbm, src_hbm, dst_hbm, *refs):
    if with_cnt:
      (out_lo_hbm, out_hi_hbm, cnt_hbm, src_v, dst_v, rows, zbuf, aggm,
       *rest) = refs
      sems, (ones_v, z16, cntm) = rest[:2 * _NB], rest[2 * _NB:]
    else:
      (out_lo_hbm, out_hi_hbm, src_v, dst_v, rows, zbuf, aggm,
       *sems) = refs
    gsem = sems[:_NB]
    ssem = sems[_NB:2 * _NB]
    c = lax.axis_index("c")
    s = lax.axis_index("s")
    w = c * _NS + s

    def zrow(r, _):
      for j in range(_HH // 16):
        zbuf[r, pl.ds(j * 16, 16)] = jnp.zeros((16,), jnp.float32)
      return 0
    lax.fori_loop(0, _ZR, zrow, 0)
    if with_cnt:
      def orow(r, _):
        ones_v[r, :] = jnp.ones((16,), jnp.float32)
        return 0
      lax.fori_loop(0, _K, orow, 0)
      def z16row(r, _):
        z16[r, :] = jnp.zeros((16,), jnp.float32)
        return 0
      lax.fori_loop(0, _ZR, z16row, 0)

    pltpu.sync_copy(src_hbm.at[w], src_v)
    pltpu.sync_copy(dst_hbm.at[w], dst_v)

    for half, (hn_hbm, out_hbm) in enumerate(
        ((hn_lo_hbm, out_lo_hbm), (hn_hi_hbm, out_hi_hbm))):
      do_cnt = with_cnt and half == 0
      for t in range(_RPS // _ZR):
        pltpu.sync_copy(zbuf, aggm.at[pl.ds(s * _RPS + t * _ZR, _ZR)])
      if do_cnt:
        for t in range(_RPS // _ZR):
          pltpu.sync_copy(z16, cntm.at[pl.ds(s * _RPS + t * _ZR, _ZR)])
      plsc.subcore_barrier()

      def gath(ch, b):
        return pltpu.make_async_copy(hn_hbm.at[src_v.at[ch]], rows.at[b],
                                     gsem[b])

      def scat(ch, b):
        return pltpu.make_async_copy(rows.at[b], aggm.at[dst_v.at[ch]],
                                     ssem[b])

      for b in range(_NB // 2):
        gath(b, b).start()

      def body(i, _):
        g0 = i * _NB
        for b in range(_NB):
          ch = g0 + b
          gath(ch, b).wait()
          scat(ch, b).start(add=True)
          if do_cnt:
            pltpu.sync_copy(ones_v, cntm.at[dst_v.at[ch]], add=True)
          nxt = ch + _NB // 2
          nb = (b + _NB // 2) % _NB

          @pl.when(nxt < _CPT)
          def _start():
            @pl.when(nxt >= _NB)
            def _drain():
              scat(nxt - _NB, nb).wait()
            gath(nxt, nb).start()
        return 0
      lax.fori_loop(0, _CPT // _NB, body, 0)

      for ch in range(_CPT - _NB, _CPT):
        scat(ch, ch % _NB).wait()
      plsc.subcore_barrier()

      pltpu.sync_copy(aggm.at[pl.ds(s * _RPS, _RPS)],
                      out_hbm.at[c, pl.ds(s * _RPS, _RPS)])
      if do_cnt:
        pltpu.sync_copy(cntm.at[pl.ds(s * _RPS, _RPS)],
                        cnt_hbm.at[c, pl.ds(s * _RPS, _RPS)])

  return agg


_sc_agg_cnt = _make_sc_agg(True)
_sc_agg = _make_sc_agg(False)


def _dot(a, b):
  return jnp.dot(a, b, preferred_element_type=jnp.float32)


def _lin0_body(x_ref, wn_ref, wr_ref, b_ref, hnl_ref, hnh_ref, hr_ref):
  xb = x_ref[...]
  hn = _dot(xb, wn_ref[...])
  hnl_ref[...] = hn[:, :_HH]
  hnh_ref[...] = hn[:, _HH:]
  hr_ref[...] = _dot(xb, wr_ref[...]) + b_ref[...]


def _combine(pl_ref, ph_ref, c_ref, hr_ref):
  agg = jnp.concatenate([pl_ref[0] + pl_ref[1], ph_ref[0] + ph_ref[1]],
                        axis=1)
  cnt = c_ref[0] + c_ref[1]
  inv = 1.0 / jnp.maximum(cnt[:, 0:1], 1.0)
  return jnp.maximum(agg * inv + hr_ref[...], 0.0)


def _comb_body(pl_ref, ph_ref, c_ref, hr_ref, wn_ref, wr_ref, b_ref,
               hnl_ref, hnh_ref, hro_ref):
  h = _combine(pl_ref, ph_ref, c_ref, hr_ref)
  hn = _dot(h, wn_ref[...])
  hnl_ref[...] = hn[:, :_HH]
  hnh_ref[...] = hn[:, _HH:]
  hro_ref[...] = _dot(h, wr_ref[...]) + b_ref[...]


def _pool_body(pl_ref, ph_ref, c_ref, hr_ref, batch_ref, w1_ref, b1_ref,
               w2_ref, b2_ref, out_ref, sums_ref, cnts_ref):
  i = pl.program_id(0)
  h = _combine(pl_ref, ph_ref, c_ref, hr_ref)
  seg = batch_ref[...]
  m = (seg == lax.broadcasted_iota(jnp.int32, (_RB, _G), 1)
       ).astype(jnp.float32)
  dn = (((0,), (0,)), ((), ()))
  sums_blk = lax.dot_general(m, h, dimension_numbers=dn,
                             preferred_element_type=jnp.float32)
  cnts_blk = lax.dot_general(m, jnp.ones_like(h), dimension_numbers=dn,
                             preferred_element_type=jnp.float32)

  @pl.when(i == 0)
  def _init():
    sums_ref[...] = jnp.zeros_like(sums_ref)
    cnts_ref[...] = jnp.zeros_like(cnts_ref)
  sums_ref[...] += sums_blk
  cnts_ref[...] += cnts_blk

  @pl.when(i == _N // _RB - 1)
  def _head():
    pooled = sums_ref[...] / jnp.maximum(cnts_ref[...], 1.0)
    t = jnp.maximum(_dot(pooled, w1_ref[...]) + b1_ref[...], 0.0)
    out_ref[...] = _dot(t, w2_ref[...]) + b2_ref[...]


def _row_spec(shape):
  if len(shape) == 2:
    return pl.BlockSpec((_RB, shape[1]), lambda i: (i, 0))
  return pl.BlockSpec((shape[0], _RB, shape[2]), lambda i: (0, i, 0))


def _full_spec(shape):
  zero = tuple(0 for _ in shape)
  return pl.BlockSpec(shape, lambda i=0, _z=zero: _z)


_HN_SHAPES = [jax.ShapeDtypeStruct((_N, _HH), jnp.float32)] * 2


def _lin0(x, wn, wr, b):
  return pl.pallas_call(
      _lin0_body,
      grid=(_N // _RB,),
      in_specs=[_row_spec((_N, _D)), _full_spec((_D, _H)),
                _full_spec((_D, _H)), _full_spec((1, _H))],
      out_specs=[_row_spec((_N, _HH)), _row_spec((_N, _HH)),
                 _row_spec((_N, _H))],
      out_shape=_HN_SHAPES + [jax.ShapeDtypeStruct((_N, _H), jnp.float32)],
  )(x, wn, wr, b)


def _comb(plo, phi, c, hr, wn, wr, b):
  return pl.pallas_call(
      _comb_body,
      grid=(_N // _RB,),
      in_specs=[_row_spec((_NC, _NP, _HH)), _row_spec((_NC, _NP, _HH)),
                _row_spec((_NC, _NP, 16)), _row_spec((_N, _H)),
                _full_spec((_H, _H)), _full_spec((_H, _H)),
                _full_spec((1, _H))],
      out_specs=[_row_spec((_N, _HH)), _row_spec((_N, _HH)),
                 _row_spec((_N, _H))],
      out_shape=_HN_SHAPES + [jax.ShapeDtypeStruct((_N, _H), jnp.float32)],
  )(plo, phi, c, hr, wn, wr, b)


def _pool(plo, phi, c, hr, batch2, w1, b1, w2, b2):
  return pl.pallas_call(
      _pool_body,
      grid=(_N // _RB,),
      in_specs=[_row_spec((_NC, _NP, _HH)), _row_spec((_NC, _NP, _HH)),
                _row_spec((_NC, _NP, 16)), _row_spec((_N, _H)),
                _row_spec((_N, 1)), _full_spec((_H, _H)),
                _full_spec((1, _H)), _full_spec((_H, _OUT)),
                _full_spec((1, _OUT))],
      out_specs=_full_spec((_G, _OUT)),
      out_shape=jax.ShapeDtypeStruct((_G, _OUT), jnp.float32),
      scratch_shapes=[pltpu.VMEM((_G, _H), jnp.float32),
                      pltpu.VMEM((_G, _H), jnp.float32)],
  )(plo, phi, c, hr, batch2, w1, b1, w2, b2)


def kernel(x, edge_index, batch, W_neigh_0, W_root_0, b_0, W_neigh_1,
           W_root_1, b_1, W_neigh_2, W_root_2, b_2, W_neigh_3, W_root_3,
           b_3, fc1_W, fc1_b, fc2_W, fc2_b):
  npad = _EP - _E
  pad_src = jnp.arange(npad, dtype=jnp.int32) % _N
  pad_dst = _N + (jnp.arange(npad, dtype=jnp.int32) % (_NP - _N))
  src = jnp.concatenate([edge_index[0], pad_src]).reshape(_NW, _CPT, _K)
  dst = jnp.concatenate([edge_index[1], pad_dst]).reshape(_NW, _CPT, _K)
  batch2 = batch.reshape(_N, 1)

  hnl, hnh, hr = _lin0(x, W_neigh_0, W_root_0, b_0.reshape(1, _H))
  plo, phi, cnt = _sc_agg_cnt(hnl, hnh, src, dst)
  hnl, hnh, hr = _comb(plo, phi, cnt, hr, W_neigh_1, W_root_1,
                       b_1.reshape(1, _H))
  plo, phi = _sc_agg(hnl, hnh, src, dst)
  hnl, hnh, hr = _comb(plo, phi, cnt, hr, W_neigh_2, W_root_2,
                       b_2.reshape(1, _H))
  plo, phi = _sc_agg(hnl, hnh, src, dst)
  hnl, hnh, hr = _comb(plo, phi, cnt, hr, W_neigh_3, W_root_3,
                       b_3.reshape(1, _H))
  plo, phi = _sc_agg(hnl, hnh, src, dst)
  return _pool(plo, phi, cnt, hr, batch2, fc1_W, fc1_b.reshape(1, _H),
               fc2_W, fc2_b.reshape(1, _OUT))


# speedup vs baseline: 9.1801x; 1.0125x over previous
import functools

import jax
import jax.numpy as jnp
from jax import lax
from jax.experimental import pallas as pl
from jax.experimental.pallas import tpu as pltpu
from jax.experimental.pallas import tpu_sc as plsc

_N, _E, _D, _H, _OUT, _G = 10000, 320000, 128, 128, 64, 128
_HH = _H // 2
_NC, _NS = 2, 16
_NW = _NC * _NS
_K = 128
_EP = 327680
_CPT = _EP // _NW // _K
_NP = 10240
_RPS = _NP // _NS
_ZR = 128
_RB = 2000
_NB = 4


def _make_sc_agg(with_cnt):
  mesh = plsc.VectorSubcoreMesh(core_axis_name="c", subcore_axis_name="s")
  out_type = [jax.ShapeDtypeStruct((_NC, _NP, _HH), jnp.float32)] * 2
  scratch = [
      pltpu.VMEM((_CPT, _K), jnp.int32),
      pltpu.VMEM((_CPT, _K), jnp.int32),
      pltpu.VMEM((_NB, _K, _HH), jnp.float32),
      pltpu.VMEM((_ZR, _HH), jnp.float32),
      pltpu.VMEM_SHARED((_NP, _HH), jnp.float32),
  ] + [pltpu.SemaphoreType.DMA] * (2 * _NB)
  if with_cnt:
    out_type.append(jax.ShapeDtypeStruct((_NC, _NP, 16), jnp.float32))
    scratch += [
        pltpu.VMEM((_K, 16), jnp.float32),
        pltpu.VMEM((_ZR, 16), jnp.float32),
        pltpu.VMEM_SHARED((_NP, 16), jnp.float32),
    ]

  @functools.partial(
      pl.kernel, mesh=mesh, out_type=out_type, scratch_types=scratch,
      compiler_params=pltpu.CompilerParams(use_tc_tiling_on_sc=False))
  def agg(hn_lo_hbm, hn_hi_hbm, src_hbm, dst_hbm, *refs):
    if with_cnt:
      (out_lo_hbm, out_hi_hbm, cnt_hbm, src_v, dst_v, rows, zbuf, aggm,
       *rest) = refs
      sems, (ones_v, z16, cntm) = rest[:2 * _NB], rest[2 * _NB:]
    else:
      (out_lo_hbm, out_hi_hbm, src_v, dst_v, rows, zbuf, aggm,
       *sems) = refs
    gsem = sems[:_NB]
    ssem = sems[_NB:2 * _NB]
    c = lax.axis_index("c")
    s = lax.axis_index("s")
    w = c * _NS + s

    def zrow(r, _):
      for j in range(_HH // 16):
        zbuf[r, pl.ds(j * 16, 16)] = jnp.zeros((16,), jnp.float32)
      return 0
    lax.fori_loop(0, _ZR, zrow, 0)
    if with_cnt:
      def orow(r, _):
        ones_v[r, :] = jnp.ones((16,), jnp.float32)
        return 0
      lax.fori_loop(0, _K, orow, 0)
      def z16row(r, _):
        z16[r, :] = jnp.zeros((16,), jnp.float32)
        return 0
      lax.fori_loop(0, _ZR, z16row, 0)

    pltpu.sync_copy(src_hbm.at[w], src_v)
    pltpu.sync_copy(dst_hbm.at[w], dst_v)

    for half, (hn_hbm, out_hbm) in enumerate(
        ((hn_lo_hbm, out_lo_hbm), (hn_hi_hbm, out_hi_hbm))):
      do_cnt = with_cnt and half == 0
      for t in range(_RPS // _ZR):
        pltpu.sync_copy(zbuf, aggm.at[pl.ds(s * _RPS + t * _ZR, _ZR)])
      if do_cnt:
        for t in range(_RPS // _ZR):
          pltpu.sync_copy(z16, cntm.at[pl.ds(s * _RPS + t * _ZR, _ZR)])
      plsc.subcore_barrier()

      def gath(ch, b):
        return pltpu.make_async_copy(hn_hbm.at[src_v.at[ch]], rows.at[b],
                                     gsem[b])

      def scat(ch, b):
        return pltpu.make_async_copy(rows.at[b], aggm.at[dst_v.at[ch]],
                                     ssem[b])

      for b in range(_NB // 2):
        gath(b, b).start()

      def body(i, _):
        g0 = i * _NB
        for b in range(_NB):
          ch = g0 + b
          gath(ch, b).wait()
          scat(ch, b).start(add=True)
          if do_cnt:
            pltpu.sync_copy(ones_v, cntm.at[dst_v.at[ch]], add=True)
          nxt = ch + _NB // 2
          nb = (b + _NB // 2) % _NB

          @pl.when(nxt < _CPT)
          def _start():
            @pl.when(nxt >= _NB)
            def _drain():
              scat(nxt - _NB, nb).wait()
            gath(nxt, nb).start()
        return 0
      lax.fori_loop(0, _CPT // _NB, body, 0)

      for ch in range(_CPT - _NB, _CPT):
        scat(ch, ch % _NB).wait()
      plsc.subcore_barrier()

      pltpu.sync_copy(aggm.at[pl.ds(s * _RPS, _RPS)],
                      out_hbm.at[c, pl.ds(s * _RPS, _RPS)])
      if do_cnt:
        pltpu.sync_copy(cntm.at[pl.ds(s * _RPS, _RPS)],
                        cnt_hbm.at[c, pl.ds(s * _RPS, _RPS)])

  return agg


_sc_agg_cnt = _make_sc_agg(True)
_sc_agg = _make_sc_agg(False)


def _dot(a, b):
  return jnp.dot(a, b, preferred_element_type=jnp.float32)


def _lin0_body(x_ref, wn_ref, wr_ref, b_ref, hnl_ref, hnh_ref, hr_ref):
  xb = x_ref[...]
  hn = _dot(xb, wn_ref[...])
  hnl_ref[...] = hn[:, :_HH]
  hnh_ref[...] = hn[:, _HH:]
  hr_ref[...] = _dot(xb, wr_ref[...]) + b_ref[...]


def _combine(pl_ref, ph_ref, c_ref, hr_ref):
  agg = jnp.concatenate([pl_ref[0] + pl_ref[1], ph_ref[0] + ph_ref[1]],
                        axis=1)
  cnt = c_ref[0] + c_ref[1]
  inv = 1.0 / jnp.maximum(cnt[:, 0:1], 1.0)
  return jnp.maximum(agg * inv + hr_ref[...], 0.0)


def _comb_body(pl_ref, ph_ref, c_ref, hr_ref, wn_ref, wr_ref, b_ref,
               hnl_ref, hnh_ref, hro_ref):
  h = _combine(pl_ref, ph_ref, c_ref, hr_ref)
  hn = _dot(h, wn_ref[...])
  hnl_ref[...] = hn[:, :_HH]
  hnh_ref[...] = hn[:, _HH:]
  hro_ref[...] = _dot(h, wr_ref[...]) + b_ref[...]


def _pool_body(pl_ref, ph_ref, c_ref, hr_ref, batch_ref, w1_ref, b1_ref,
               w2_ref, b2_ref, out_ref, sums_ref, cnts_ref):
  i = pl.program_id(0)
  h = _combine(pl_ref, ph_ref, c_ref, hr_ref)
  seg = batch_ref[...]
  m = (seg == lax.broadcasted_iota(jnp.int32, (_RB, _G), 1)
       ).astype(jnp.float32)
  dn = (((0,), (0,)), ((), ()))
  sums_blk = lax.dot_general(m, h, dimension_numbers=dn,
                             preferred_element_type=jnp.float32)
  cnts_blk = lax.dot_general(m, jnp.ones_like(h), dimension_numbers=dn,
                             preferred_element_type=jnp.float32)

  @pl.when(i == 0)
  def _init():
    sums_ref[...] = jnp.zeros_like(sums_ref)
    cnts_ref[...] = jnp.zeros_like(cnts_ref)
  sums_ref[...] += sums_blk
  cnts_ref[...] += cnts_blk

  @pl.when(i == _N // _RB - 1)
  def _head():
    pooled = sums_ref[...] / jnp.maximum(cnts_ref[...], 1.0)
    t = jnp.maximum(_dot(pooled, w1_ref[...]) + b1_ref[...], 0.0)
    out_ref[...] = _dot(t, w2_ref[...]) + b2_ref[...]


def _row_spec(shape):
  if len(shape) == 2:
    return pl.BlockSpec((_RB, shape[1]), lambda i: (i, 0))
  return pl.BlockSpec((shape[0], _RB, shape[2]), lambda i: (0, i, 0))


def _full_spec(shape):
  zero = tuple(0 for _ in shape)
  return pl.BlockSpec(shape, lambda i=0, _z=zero: _z)


_HN_SHAPES = [jax.ShapeDtypeStruct((_N, _HH), jnp.float32)] * 2


def _lin0(x, wn, wr, b):
  return pl.pallas_call(
      _lin0_body,
      grid=(_N // _RB,),
      in_specs=[_row_spec((_N, _D)), _full_spec((_D, _H)),
                _full_spec((_D, _H)), _full_spec((1, _H))],
      out_specs=[_row_spec((_N, _HH)), _row_spec((_N, _HH)),
                 _row_spec((_N, _H))],
      out_shape=_HN_SHAPES + [jax.ShapeDtypeStruct((_N, _H), jnp.float32)],
  )(x, wn, wr, b)


def _comb(plo, phi, c, hr, wn, wr, b):
  return pl.pallas_call(
      _comb_body,
      grid=(_N // _RB,),
      in_specs=[_row_spec((_NC, _NP, _HH)), _row_spec((_NC, _NP, _HH)),
                _row_spec((_NC, _NP, 16)), _row_spec((_N, _H)),
                _full_spec((_H, _H)), _full_spec((_H, _H)),
                _full_spec((1, _H))],
      out_specs=[_row_spec((_N, _HH)), _row_spec((_N, _HH)),
                 _row_spec((_N, _H))],
      out_shape=_HN_SHAPES + [jax.ShapeDtypeStruct((_N, _H), jnp.float32)],
  )(plo, phi, c, hr, wn, wr, b)


def _pool(plo, phi, c, hr, batch2, w1, b1, w2, b2):
  return pl.pallas_call(
      _pool_body,
      grid=(_N // _RB,),
      in_specs=[_row_spec((_NC, _NP, _HH)), _row_spec((_NC, _NP, _HH)),
                _row_spec((_NC, _NP, 16)), _row_spec((_N, _H)),
                _row_spec((_N, 1)), _full_spec((_H, _H)),
                _full_spec((1, _H)), _full_spec((_H, _OUT)),
                _full_spec((1, _OUT))],
      out_specs=_full_spec((_G, _OUT)),
      out_shape=jax.ShapeDtypeStruct((_G, _OUT), jnp.float32),
      scratch_shapes=[pltpu.VMEM((_G, _H), jnp.float32),
                      pltpu.VMEM((_G, _H), jnp.float32)],
  )(plo, phi, c, hr, batch2, w1, b1, w2, b2)


def kernel(x, edge_index, batch, W_neigh_0, W_root_0, b_0, W_neigh_1,
           W_root_1, b_1, W_neigh_2, W_root_2, b_2, W_neigh_3, W_root_3,
           b_3, fc1_W, fc1_b, fc2_W, fc2_b):
  npad = _EP - _E
  pad_src = jnp.arange(npad, dtype=jnp.int32) % _N
  pad_dst = _N + (jnp.arange(npad, dtype=jnp.int32) % (_NP - _N))
  src = jnp.concatenate([edge_index[0], pad_src]).reshape(_NW, _CPT, _K)
  dst = jnp.concatenate([edge_index[1], pad_dst]).reshape(_NW, _CPT, _K)
  batch2 = batch.reshape(_N, 1)

  hnl, hnh, hr = _lin0(x, W_neigh_0, W_root_0, b_0.reshape(1, _H))
  plo, phi, cnt = _sc_agg_cnt(hnl, hnh, src, dst)
  hnl, hnh, hr = _comb(plo, phi, cnt, hr, W_neigh_1, W_root_1,
                       b_1.reshape(1, _H))
  plo, phi = _sc_agg(hnl, hnh, src, dst)
  hnl, hnh, hr = _comb(plo, phi, cnt, hr, W_neigh_2, W_root_2,
                       b_2.reshape(1, _H))
  plo, phi = _sc_agg(hnl, hnh, src, dst)
  hnl, hnh, hr = _comb(plo, phi, cnt, hr, W_neigh_3, W_root_3,
                       b_3.reshape(1, _H))
  plo, phi = _sc_agg(hnl, hnh, src, dst)
  return _pool(plo, phi, cnt, hr, batch2, fc1_W, fc1_b.reshape(1, _H),
               fc2_W, fc2_b.reshape(1, _OUT))


# speedup vs baseline: 9.5914x; 1.0448x over previous
import functools

import jax
import jax.numpy as jnp
from jax import lax
from jax.experimental import pallas as pl
from jax.experimental.pallas import tpu as pltpu
from jax.experimental.pallas import tpu_sc as plsc

_N, _E, _D, _H, _OUT, _G = 10000, 320000, 128, 128, 64, 128
_HH = _H // 2
_NC, _NS = 2, 16
_NW = _NC * _NS
_K = 64
_EP = 327680
_CPT = _EP // _NW // _K
_NP = 10240
_RPS = _NP // _NS
_ZR = 128
_RB = 2000
_NB = 8


def _make_sc_agg(with_cnt):
  mesh = plsc.VectorSubcoreMesh(core_axis_name="c", subcore_axis_name="s")
  out_type = [jax.ShapeDtypeStruct((_NC, _NP, _HH), jnp.float32)] * 2
  scratch = [
      pltpu.VMEM((_CPT, _K), jnp.int32),
      pltpu.VMEM((_CPT, _K), jnp.int32),
      pltpu.VMEM((_NB, _K, _HH), jnp.float32),
      pltpu.VMEM((_ZR, _HH), jnp.float32),
      pltpu.VMEM_SHARED((_NP, _HH), jnp.float32),
  ] + [pltpu.SemaphoreType.DMA] * (2 * _NB)
  if with_cnt:
    out_type.append(jax.ShapeDtypeStruct((_NC, _NP, 16), jnp.float32))
    scratch += [
        pltpu.VMEM((_K, 16), jnp.float32),
        pltpu.VMEM((_ZR, 16), jnp.float32),
        pltpu.VMEM_SHARED((_NP, 16), jnp.float32),
    ]

  @functools.partial(
      pl.kernel, mesh=mesh, out_type=out_type, scratch_types=scratch,
      compiler_params=pltpu.CompilerParams(use_tc_tiling_on_sc=False))
  def agg(hn_lo_hbm, hn_hi_hbm, src_hbm, dst_hbm, *refs):
    if with_cnt:
      (out_lo_hbm, out_hi_hbm, cnt_hbm, src_v, dst_v, rows, zbuf, aggm,
       *rest) = refs
      sems, (ones_v, z16, cntm) = rest[:2 * _NB], rest[2 * _NB:]
    else:
      (out_lo_hbm, out_hi_hbm, src_v, dst_v, rows, zbuf, aggm,
       *sems) = refs
    gsem = sems[:_NB]
    ssem = sems[_NB:2 * _NB]
    c = lax.axis_index("c")
    s = lax.axis_index("s")
    w = c * _NS + s

    def zrow(r, _):
      for j in range(_HH // 16):
        zbuf[r, pl.ds(j * 16, 16)] = jnp.zeros((16,), jnp.float32)
      return 0
    lax.fori_loop(0, _ZR, zrow, 0)
    if with_cnt:
      def orow(r, _):
        ones_v[r, :] = jnp.ones((16,), jnp.float32)
        return 0
      lax.fori_loop(0, _K, orow, 0)
      def z16row(r, _):
        z16[r, :] = jnp.zeros((16,), jnp.float32)
        return 0
      lax.fori_loop(0, _ZR, z16row, 0)

    pltpu.sync_copy(src_hbm.at[w], src_v)
    pltpu.sync_copy(dst_hbm.at[w], dst_v)

    for half, (hn_hbm, out_hbm) in enumerate(
        ((hn_lo_hbm, out_lo_hbm), (hn_hi_hbm, out_hi_hbm))):
      do_cnt = with_cnt and half == 0
      for t in range(_RPS // _ZR):
        pltpu.sync_copy(zbuf, aggm.at[pl.ds(s * _RPS + t * _ZR, _ZR)])
      if do_cnt:
        for t in range(_RPS // _ZR):
          pltpu.sync_copy(z16, cntm.at[pl.ds(s * _RPS + t * _ZR, _ZR)])
      plsc.subcore_barrier()

      def gath(ch, b):
        return pltpu.make_async_copy(hn_hbm.at[src_v.at[ch]], rows.at[b],
                                     gsem[b])

      def scat(ch, b):
        return pltpu.make_async_copy(rows.at[b], aggm.at[dst_v.at[ch]],
                                     ssem[b])

      for b in range(_NB // 2):
        gath(b, b).start()

      def body(i, _):
        g0 = i * _NB
        for b in range(_NB):
          ch = g0 + b
          gath(ch, b).wait()
          scat(ch, b).start(add=True)
          if do_cnt:
            pltpu.sync_copy(ones_v, cntm.at[dst_v.at[ch]], add=True)
          nxt = ch + _NB // 2
          nb = (b + _NB // 2) % _NB

          @pl.when(nxt < _CPT)
          def _start():
            @pl.when(nxt >= _NB)
            def _drain():
              scat(nxt - _NB, nb).wait()
            gath(nxt, nb).start()
        return 0
      lax.fori_loop(0, _CPT // _NB, body, 0)

      for ch in range(_CPT - _NB, _CPT):
        scat(ch, ch % _NB).wait()
      plsc.subcore_barrier()

      pltpu.sync_copy(aggm.at[pl.ds(s * _RPS, _RPS)],
                      out_hbm.at[c, pl.ds(s * _RPS, _RPS)])
      if do_cnt:
        pltpu.sync_copy(cntm.at[pl.ds(s * _RPS, _RPS)],
                        cnt_hbm.at[c, pl.ds(s * _RPS, _RPS)])

  return agg


_sc_agg_cnt = _make_sc_agg(True)
_sc_agg = _make_sc_agg(False)


def _dot(a, b):
  return jnp.dot(a, b, preferred_element_type=jnp.float32)


def _lin0_body(x_ref, wn_ref, wr_ref, b_ref, hnl_ref, hnh_ref, hr_ref):
  xb = x_ref[...]
  hn = _dot(xb, wn_ref[...])
  hnl_ref[...] = hn[:, :_HH]
  hnh_ref[...] = hn[:, _HH:]
  hr_ref[...] = _dot(xb, wr_ref[...]) + b_ref[...]


def _combine(pl_ref, ph_ref, c_ref, hr_ref):
  agg = jnp.concatenate([pl_ref[0] + pl_ref[1], ph_ref[0] + ph_ref[1]],
                        axis=1)
  cnt = c_ref[0] + c_ref[1]
  inv = 1.0 / jnp.maximum(cnt[:, 0:1], 1.0)
  return jnp.maximum(agg * inv + hr_ref[...], 0.0)


def _comb_body(pl_ref, ph_ref, c_ref, hr_ref, wn_ref, wr_ref, b_ref,
               hnl_ref, hnh_ref, hro_ref):
  h = _combine(pl_ref, ph_ref, c_ref, hr_ref)
  hn = _dot(h, wn_ref[...])
  hnl_ref[...] = hn[:, :_HH]
  hnh_ref[...] = hn[:, _HH:]
  hro_ref[...] = _dot(h, wr_ref[...]) + b_ref[...]


def _pool_body(pl_ref, ph_ref, c_ref, hr_ref, batch_ref, w1_ref, b1_ref,
               w2_ref, b2_ref, out_ref, sums_ref, cnts_ref):
  i = pl.program_id(0)
  h = _combine(pl_ref, ph_ref, c_ref, hr_ref)
  seg = batch_ref[...]
  m = (seg == lax.broadcasted_iota(jnp.int32, (_RB, _G), 1)
       ).astype(jnp.float32)
  dn = (((0,), (0,)), ((), ()))
  sums_blk = lax.dot_general(m, h, dimension_numbers=dn,
                             preferred_element_type=jnp.float32)
  cnts_blk = lax.dot_general(m, jnp.ones_like(h), dimension_numbers=dn,
                             preferred_element_type=jnp.float32)

  @pl.when(i == 0)
  def _init():
    sums_ref[...] = jnp.zeros_like(sums_ref)
    cnts_ref[...] = jnp.zeros_like(cnts_ref)
  sums_ref[...] += sums_blk
  cnts_ref[...] += cnts_blk

  @pl.when(i == _N // _RB - 1)
  def _head():
    pooled = sums_ref[...] / jnp.maximum(cnts_ref[...], 1.0)
    t = jnp.maximum(_dot(pooled, w1_ref[...]) + b1_ref[...], 0.0)
    out_ref[...] = _dot(t, w2_ref[...]) + b2_ref[...]


def _row_spec(shape):
  if len(shape) == 2:
    return pl.BlockSpec((_RB, shape[1]), lambda i: (i, 0))
  return pl.BlockSpec((shape[0], _RB, shape[2]), lambda i: (0, i, 0))


def _full_spec(shape):
  zero = tuple(0 for _ in shape)
  return pl.BlockSpec(shape, lambda i=0, _z=zero: _z)


_HN_SHAPES = [jax.ShapeDtypeStruct((_N, _HH), jnp.float32)] * 2


def _lin0(x, wn, wr, b):
  return pl.pallas_call(
      _lin0_body,
      grid=(_N // _RB,),
      in_specs=[_row_spec((_N, _D)), _full_spec((_D, _H)),
                _full_spec((_D, _H)), _full_spec((1, _H))],
      out_specs=[_row_spec((_N, _HH)), _row_spec((_N, _HH)),
                 _row_spec((_N, _H))],
      out_shape=_HN_SHAPES + [jax.ShapeDtypeStruct((_N, _H), jnp.float32)],
  )(x, wn, wr, b)


def _comb(plo, phi, c, hr, wn, wr, b):
  return pl.pallas_call(
      _comb_body,
      grid=(_N // _RB,),
      in_specs=[_row_spec((_NC, _NP, _HH)), _row_spec((_NC, _NP, _HH)),
                _row_spec((_NC, _NP, 16)), _row_spec((_N, _H)),
                _full_spec((_H, _H)), _full_spec((_H, _H)),
                _full_spec((1, _H))],
      out_specs=[_row_spec((_N, _HH)), _row_spec((_N, _HH)),
                 _row_spec((_N, _H))],
      out_shape=_HN_SHAPES + [jax.ShapeDtypeStruct((_N, _H), jnp.float32)],
  )(plo, phi, c, hr, wn, wr, b)


def _pool(plo, phi, c, hr, batch2, w1, b1, w2, b2):
  return pl.pallas_call(
      _pool_body,
      grid=(_N // _RB,),
      in_specs=[_row_spec((_NC, _NP, _HH)), _row_spec((_NC, _NP, _HH)),
                _row_spec((_NC, _NP, 16)), _row_spec((_N, _H)),
                _row_spec((_N, 1)), _full_spec((_H, _H)),
                _full_spec((1, _H)), _full_spec((_H, _OUT)),
                _full_spec((1, _OUT))],
      out_specs=_full_spec((_G, _OUT)),
      out_shape=jax.ShapeDtypeStruct((_G, _OUT), jnp.float32),
      scratch_shapes=[pltpu.VMEM((_G, _H), jnp.float32),
                      pltpu.VMEM((_G, _H), jnp.float32)],
  )(plo, phi, c, hr, batch2, w1, b1, w2, b2)


def kernel(x, edge_index, batch, W_neigh_0, W_root_0, b_0, W_neigh_1,
           W_root_1, b_1, W_neigh_2, W_root_2, b_2, W_neigh_3, W_root_3,
           b_3, fc1_W, fc1_b, fc2_W, fc2_b):
  npad = _EP - _E
  pad_src = jnp.arange(npad, dtype=jnp.int32) % _N
  pad_dst = _N + (jnp.arange(npad, dtype=jnp.int32) % (_NP - _N))
  src = jnp.concatenate([edge_index[0], pad_src]).reshape(_NW, _CPT, _K)
  dst = jnp.concatenate([edge_index[1], pad_dst]).reshape(_NW, _CPT, _K)
  batch2 = batch.reshape(_N, 1)

  hnl, hnh, hr = _lin0(x, W_neigh_0, W_root_0, b_0.reshape(1, _H))
  plo, phi, cnt = _sc_agg_cnt(hnl, hnh, src, dst)
  hnl, hnh, hr = _comb(plo, phi, cnt, hr, W_neigh_1, W_root_1,
                       b_1.reshape(1, _H))
  plo, phi = _sc_agg(hnl, hnh, src, dst)
  hnl, hnh, hr = _comb(plo, phi, cnt, hr, W_neigh_2, W_root_2,
                       b_2.reshape(1, _H))
  plo, phi = _sc_agg(hnl, hnh, src, dst)
  hnl, hnh, hr = _comb(plo, phi, cnt, hr, W_neigh_3, W_root_3,
                       b_3.reshape(1, _H))
  plo, phi = _sc_agg(hnl, hnh, src, dst)
  return _pool(plo, phi, cnt, hr, batch2, fc1_W, fc1_b.reshape(1, _H),
               fc2_W, fc2_b.reshape(1, _OUT))


# speedup vs baseline: 9.9225x; 1.0345x over previous
import functools

import jax
import jax.numpy as jnp
from jax import lax
from jax.experimental import pallas as pl
from jax.experimental.pallas import tpu as pltpu
from jax.experimental.pallas import tpu_sc as plsc

_N, _E, _D, _H, _OUT, _G = 10000, 320000, 128, 128, 64, 128
_HH = _H // 2
_NC, _NS = 2, 16
_NW = _NC * _NS
_K = 64
_EP = 327680
_CPT = _EP // _NW // _K
_NP = 10240
_RPS = _NP // _NS
_ZR = 128
_RB = 2000
_NB = 10


def _make_sc_agg(with_cnt):
  mesh = plsc.VectorSubcoreMesh(core_axis_name="c", subcore_axis_name="s")
  out_type = [jax.ShapeDtypeStruct((_NC, _NP, _HH), jnp.float32)] * 2
  scratch = [
      pltpu.VMEM((_CPT, _K), jnp.int32),
      pltpu.VMEM((_CPT, _K), jnp.int32),
      pltpu.VMEM((_NB, _K, _HH), jnp.float32),
      pltpu.VMEM((_ZR, _HH), jnp.float32),
      pltpu.VMEM_SHARED((_NP, _HH), jnp.float32),
  ] + [pltpu.SemaphoreType.DMA] * (2 * _NB)
  if with_cnt:
    out_type.append(jax.ShapeDtypeStruct((_NC, _NP, 16), jnp.float32))
    scratch += [
        pltpu.VMEM((_K, 16), jnp.float32),
        pltpu.VMEM((_ZR, 16), jnp.float32),
        pltpu.VMEM_SHARED((_NP, 16), jnp.float32),
    ]

  @functools.partial(
      pl.kernel, mesh=mesh, out_type=out_type, scratch_types=scratch,
      compiler_params=pltpu.CompilerParams(use_tc_tiling_on_sc=False))
  def agg(hn_lo_hbm, hn_hi_hbm, src_hbm, dst_hbm, *refs):
    if with_cnt:
      (out_lo_hbm, out_hi_hbm, cnt_hbm, src_v, dst_v, rows, zbuf, aggm,
       *rest) = refs
      sems, (ones_v, z16, cntm) = rest[:2 * _NB], rest[2 * _NB:]
    else:
      (out_lo_hbm, out_hi_hbm, src_v, dst_v, rows, zbuf, aggm,
       *sems) = refs
    gsem = sems[:_NB]
    ssem = sems[_NB:2 * _NB]
    c = lax.axis_index("c")
    s = lax.axis_index("s")
    w = c * _NS + s

    def zrow(r, _):
      for j in range(_HH // 16):
        zbuf[r, pl.ds(j * 16, 16)] = jnp.zeros((16,), jnp.float32)
      return 0
    lax.fori_loop(0, _ZR, zrow, 0)
    if with_cnt:
      def orow(r, _):
        ones_v[r, :] = jnp.ones((16,), jnp.float32)
        return 0
      lax.fori_loop(0, _K, orow, 0)
      def z16row(r, _):
        z16[r, :] = jnp.zeros((16,), jnp.float32)
        return 0
      lax.fori_loop(0, _ZR, z16row, 0)

    pltpu.sync_copy(src_hbm.at[w], src_v)
    pltpu.sync_copy(dst_hbm.at[w], dst_v)

    for half, (hn_hbm, out_hbm) in enumerate(
        ((hn_lo_hbm, out_lo_hbm), (hn_hi_hbm, out_hi_hbm))):
      do_cnt = with_cnt and half == 0
      for t in range(_RPS // _ZR):
        pltpu.sync_copy(zbuf, aggm.at[pl.ds(s * _RPS + t * _ZR, _ZR)])
      if do_cnt:
        for t in range(_RPS // _ZR):
          pltpu.sync_copy(z16, cntm.at[pl.ds(s * _RPS + t * _ZR, _ZR)])
      plsc.subcore_barrier()

      def gath(ch, b):
        return pltpu.make_async_copy(hn_hbm.at[src_v.at[ch]], rows.at[b],
                                     gsem[b])

      def scat(ch, b):
        return pltpu.make_async_copy(rows.at[b], aggm.at[dst_v.at[ch]],
                                     ssem[b])

      for b in range(_NB // 2):
        gath(b, b).start()

      def body(i, _):
        g0 = i * _NB
        for b in range(_NB):
          ch = g0 + b
          gath(ch, b).wait()
          scat(ch, b).start(add=True)
          if do_cnt:
            pltpu.sync_copy(ones_v, cntm.at[dst_v.at[ch]], add=True)
          nxt = ch + _NB // 2
          nb = (b + _NB // 2) % _NB

          @pl.when(nxt < _CPT)
          def _start():
            @pl.when(nxt >= _NB)
            def _drain():
              scat(nxt - _NB, nb).wait()
            gath(nxt, nb).start()
        return 0
      lax.fori_loop(0, _CPT // _NB, body, 0)

      for ch in range(_CPT - _NB, _CPT):
        scat(ch, ch % _NB).wait()
      plsc.subcore_barrier()

      pltpu.sync_copy(aggm.at[pl.ds(s * _RPS, _RPS)],
                      out_hbm.at[c, pl.ds(s * _RPS, _RPS)])
      if do_cnt:
        pltpu.sync_copy(cntm.at[pl.ds(s * _RPS, _RPS)],
                        cnt_hbm.at[c, pl.ds(s * _RPS, _RPS)])

  return agg


_sc_agg_cnt = _make_sc_agg(True)
_sc_agg = _make_sc_agg(False)


def _dot(a, b):
  return jnp.dot(a, b, preferred_element_type=jnp.float32)


def _lin0_body(x_ref, wn_ref, wr_ref, b_ref, hnl_ref, hnh_ref, hr_ref):
  xb = x_ref[...]
  hn = _dot(xb, wn_ref[...])
  hnl_ref[...] = hn[:, :_HH]
  hnh_ref[...] = hn[:, _HH:]
  hr_ref[...] = _dot(xb, wr_ref[...]) + b_ref[...]


def _combine(pl_ref, ph_ref, c_ref, hr_ref):
  agg = jnp.concatenate([pl_ref[0] + pl_ref[1], ph_ref[0] + ph_ref[1]],
                        axis=1)
  cnt = c_ref[0] + c_ref[1]
  inv = 1.0 / jnp.maximum(cnt[:, 0:1], 1.0)
  return jnp.maximum(agg * inv + hr_ref[...], 0.0)


def _comb_body(pl_ref, ph_ref, c_ref, hr_ref, wn_ref, wr_ref, b_ref,
               hnl_ref, hnh_ref, hro_ref):
  h = _combine(pl_ref, ph_ref, c_ref, hr_ref)
  hn = _dot(h, wn_ref[...])
  hnl_ref[...] = hn[:, :_HH]
  hnh_ref[...] = hn[:, _HH:]
  hro_ref[...] = _dot(h, wr_ref[...]) + b_ref[...]


def _pool_body(pl_ref, ph_ref, c_ref, hr_ref, batch_ref, w1_ref, b1_ref,
               w2_ref, b2_ref, out_ref, sums_ref, cnts_ref):
  i = pl.program_id(0)
  h = _combine(pl_ref, ph_ref, c_ref, hr_ref)
  seg = batch_ref[...]
  m = (seg == lax.broadcasted_iota(jnp.int32, (_RB, _G), 1)
       ).astype(jnp.float32)
  dn = (((0,), (0,)), ((), ()))
  sums_blk = lax.dot_general(m, h, dimension_numbers=dn,
                             preferred_element_type=jnp.float32)
  cnts_blk = lax.dot_general(m, jnp.ones_like(h), dimension_numbers=dn,
                             preferred_element_type=jnp.float32)

  @pl.when(i == 0)
  def _init():
    sums_ref[...] = jnp.zeros_like(sums_ref)
    cnts_ref[...] = jnp.zeros_like(cnts_ref)
  sums_ref[...] += sums_blk
  cnts_ref[...] += cnts_blk

  @pl.when(i == _N // _RB - 1)
  def _head():
    pooled = sums_ref[...] / jnp.maximum(cnts_ref[...], 1.0)
    t = jnp.maximum(_dot(pooled, w1_ref[...]) + b1_ref[...], 0.0)
    out_ref[...] = _dot(t, w2_ref[...]) + b2_ref[...]


def _row_spec(shape):
  if len(shape) == 2:
    return pl.BlockSpec((_RB, shape[1]), lambda i: (i, 0))
  return pl.BlockSpec((shape[0], _RB, shape[2]), lambda i: (0, i, 0))


def _full_spec(shape):
  zero = tuple(0 for _ in shape)
  return pl.BlockSpec(shape, lambda i=0, _z=zero: _z)


_HN_SHAPES = [jax.ShapeDtypeStruct((_N, _HH), jnp.float32)] * 2


def _lin0(x, wn, wr, b):
  return pl.pallas_call(
      _lin0_body,
      grid=(_N // _RB,),
      in_specs=[_row_spec((_N, _D)), _full_spec((_D, _H)),
                _full_spec((_D, _H)), _full_spec((1, _H))],
      out_specs=[_row_spec((_N, _HH)), _row_spec((_N, _HH)),
                 _row_spec((_N, _H))],
      out_shape=_HN_SHAPES + [jax.ShapeDtypeStruct((_N, _H), jnp.float32)],
  )(x, wn, wr, b)


def _comb(plo, phi, c, hr, wn, wr, b):
  return pl.pallas_call(
      _comb_body,
      grid=(_N // _RB,),
      in_specs=[_row_spec((_NC, _NP, _HH)), _row_spec((_NC, _NP, _HH)),
                _row_spec((_NC, _NP, 16)), _row_spec((_N, _H)),
                _full_spec((_H, _H)), _full_spec((_H, _H)),
                _full_spec((1, _H))],
      out_specs=[_row_spec((_N, _HH)), _row_spec((_N, _HH)),
                 _row_spec((_N, _H))],
      out_shape=_HN_SHAPES + [jax.ShapeDtypeStruct((_N, _H), jnp.float32)],
  )(plo, phi, c, hr, wn, wr, b)


def _pool(plo, phi, c, hr, batch2, w1, b1, w2, b2):
  return pl.pallas_call(
      _pool_body,
      grid=(_N // _RB,),
      in_specs=[_row_spec((_NC, _NP, _HH)), _row_spec((_NC, _NP, _HH)),
                _row_spec((_NC, _NP, 16)), _row_spec((_N, _H)),
                _row_spec((_N, 1)), _full_spec((_H, _H)),
                _full_spec((1, _H)), _full_spec((_H, _OUT)),
                _full_spec((1, _OUT))],
      out_specs=_full_spec((_G, _OUT)),
      out_shape=jax.ShapeDtypeStruct((_G, _OUT), jnp.float32),
      scratch_shapes=[pltpu.VMEM((_G, _H), jnp.float32),
                      pltpu.VMEM((_G, _H), jnp.float32)],
  )(plo, phi, c, hr, batch2, w1, b1, w2, b2)


def kernel(x, edge_index, batch, W_neigh_0, W_root_0, b_0, W_neigh_1,
           W_root_1, b_1, W_neigh_2, W_root_2, b_2, W_neigh_3, W_root_3,
           b_3, fc1_W, fc1_b, fc2_W, fc2_b):
  npad = _EP - _E
  pad_src = jnp.arange(npad, dtype=jnp.int32) % _N
  pad_dst = _N + (jnp.arange(npad, dtype=jnp.int32) % (_NP - _N))
  src = jnp.concatenate([edge_index[0], pad_src]).reshape(_NW, _CPT, _K)
  dst = jnp.concatenate([edge_index[1], pad_dst]).reshape(_NW, _CPT, _K)
  batch2 = batch.reshape(_N, 1)

  hnl, hnh, hr = _lin0(x, W_neigh_0, W_root_0, b_0.reshape(1, _H))
  plo, phi, cnt = _sc_agg_cnt(hnl, hnh, src, dst)
  hnl, hnh, hr = _comb(plo, phi, cnt, hr, W_neigh_1, W_root_1,
                       b_1.reshape(1, _H))
  plo, phi = _sc_agg(hnl, hnh, src, dst)
  hnl, hnh, hr = _comb(plo, phi, cnt, hr, W_neigh_2, W_root_2,
                       b_2.reshape(1, _H))
  plo, phi = _sc_agg(hnl, hnh, src, dst)
  hnl, hnh, hr = _comb(plo, phi, cnt, hr, W_neigh_3, W_root_3,
                       b_3.reshape(1, _H))
  plo, phi = _sc_agg(hnl, hnh, src, dst)
  return _pool(plo, phi, cnt, hr, batch2, fc1_W, fc1_b.reshape(1, _H),
               fc2_W, fc2_b.reshape(1, _OUT))
